# Initial kernel scaffold; baseline (speedup 1.0000x reference)
import functools

import jax
import jax.numpy as jnp
from jax import lax
from jax.experimental import pallas as pl
from jax.experimental.pallas import tpu as pltpu
from jax.experimental.pallas import tpu_sc as plsc

NC = 2
NS = 16
L = 16
NW = NC * NS

P_CHUNK = 8
NBUF = 2


def _pooling_sc(table, idx_chunks, w_pad, m_pad, *, B, P_pad, C, M):
  PW = P_pad // NW
  NCH = PW // P_CHUNK
  CCH = C // L
  ROWS = P_CHUNK * M

  mesh = plsc.VectorSubcoreMesh(core_axis_name="c", subcore_axis_name="s")

  @functools.partial(
      pl.kernel,
      out_type=jax.ShapeDtypeStruct((B, P_pad, C), jnp.float32),
      mesh=mesh,
      scratch_types=[
          pltpu.VMEM((NCH, ROWS), jnp.int32),
          pltpu.VMEM((PW, M), jnp.float32),
          pltpu.VMEM((PW, M), jnp.float32),
          pltpu.VMEM((NBUF, ROWS, C), jnp.float32),
          pltpu.VMEM((NBUF, P_CHUNK, C), jnp.float32),
          pltpu.VMEM((P_CHUNK, M), jnp.float32),
          pltpu.SemaphoreType.DMA,
          pltpu.SemaphoreType.DMA,
      ],
  )
  def k(table_h, idx_h, w_h, m_h, out_h,
        idx_v, w_v, m_v, rows_v, out_v, wn_v, sem0, sem1):
    sems = (sem0, sem1)
    wid = lax.axis_index("s") * NC + lax.axis_index("c")
    base_p = wid * PW

    pltpu.sync_copy(w_h.at[pl.ds(base_p, PW)], w_v)
    pltpu.sync_copy(m_h.at[pl.ds(base_p, PW)], m_v)

    def start_gather(ci, t):
      pltpu.async_copy(table_h.at[idx_v.at[ci]], rows_v.at[t], sems[t])

    def wait_gather(t):
      pltpu.make_async_copy(
          table_h.at[idx_v.at[0]], rows_v.at[t], sems[t]).wait()

    def compute_chunk(ci, t, b):
      for j in range(P_CHUNK):
        p_loc = ci * P_CHUNK + j
        wv = w_v[p_loc]
        mv = m_v[p_loc]
        pv = jnp.abs(wv) * mv
        s = jnp.sum(pv) + jnp.float32(1e-8)
        wn_v[j] = pv / s
      for j in range(P_CHUNK):
        acc = [jnp.zeros((L,), jnp.float32) for _ in range(CCH)]
        for m in range(M):
          ws = wn_v[j, m]
          for cc in range(CCH):
            acc[cc] = acc[cc] + ws * rows_v[t, j * M + m, pl.ds(cc * L, L)]
        for cc in range(CCH):
          out_v[t, j, pl.ds(cc * L, L)] = acc[cc]
      pltpu.sync_copy(
          out_v.at[t],
          out_h.at[b].at[pl.ds(base_p + ci * P_CHUNK, P_CHUNK)])

    for b in range(B):
      pltpu.sync_copy(idx_h.at[b].at[pl.ds(wid * NCH, NCH)], idx_v)
      for t in range(NBUF):
        start_gather(t, t)

      def body(i, _, b=b):
        ci0 = i * NBUF
        for t in range(NBUF):
          wait_gather(t)
          compute_chunk(ci0 + t, t, b)
          start_gather(ci0 + t + NBUF, t)
        return 0

      lax.fori_loop(0, NCH // NBUF - 1, body, 0)
      for t in range(NBUF):
        ci = NCH - NBUF + t
        wait_gather(t)
        compute_chunk(ci, t, b)

  return k(table, idx_chunks, w_pad, m_pad)


def kernel(in_pc_pad, neighbor_id_lstlst, neighbor_mask_lst, p_neighbors):
  B, IN_PN, C = in_pc_pad.shape
  OUT_PN, M = p_neighbors.shape
  assert M == L and C % L == 0

  P_pad = ((OUT_PN + NW * P_CHUNK - 1) // (NW * P_CHUNK)) * (NW * P_CHUNK)
  pad = P_pad - OUT_PN

  ids = neighbor_id_lstlst.astype(jnp.int32)
  ids = jnp.pad(ids, ((0, pad), (0, 0)))
  w_pad = jnp.pad(p_neighbors, ((0, pad), (0, 0)))
  m_pad = jnp.pad(neighbor_mask_lst, ((0, pad), (0, 0)))

  offs = (jnp.arange(B, dtype=jnp.int32) * IN_PN)[:, None, None]
  idx_chunks = (ids[None] + offs).reshape(B, P_pad * M // (P_CHUNK * M),
                                          P_CHUNK * M)
  table = in_pc_pad.reshape(B * IN_PN, C)

  out = _pooling_sc(table, idx_chunks, w_pad, m_pad,
                    B=B, P_pad=P_pad, C=C, M=M)
  return out[:, :OUT_PN, :]


# speedup vs baseline: 3.2160x; 3.2160x over previous
import functools

import jax
import jax.numpy as jnp
from jax import lax
from jax.experimental import pallas as pl
from jax.experimental.pallas import tpu as pltpu
from jax.experimental.pallas import tpu_sc as plsc

NC = 2
NS = 16
L = 16
NW = NC * NS

P_CHUNK = 8
NBUF = 2


def _pooling_sc(table, idx_chunks, w_chunks, m_chunks, *, B, P_pad, C, M):
  PW = P_pad // NW
  NCH = PW // P_CHUNK
  CCH = C // L
  ROWS = P_CHUNK * M

  mesh = plsc.VectorSubcoreMesh(core_axis_name="c", subcore_axis_name="s")

  @functools.partial(
      pl.kernel,
      out_type=jax.ShapeDtypeStruct((B, P_pad, C), jnp.float32),
      mesh=mesh,
      compiler_params=pltpu.CompilerParams(needs_layout_passes=False),
      scratch_types=[
          pltpu.VMEM((NCH, ROWS), jnp.int32),
          pltpu.VMEM((NCH, ROWS), jnp.float32),
          pltpu.VMEM((NCH, ROWS), jnp.float32),
          pltpu.VMEM((NBUF, ROWS, C), jnp.float32),
          pltpu.VMEM((NBUF, P_CHUNK, C), jnp.float32),
          pltpu.SemaphoreType.DMA,
          pltpu.SemaphoreType.DMA,
      ],
  )
  def k(table_h, idx_h, w_h, m_h, out_h,
        idx_v, w_v, m_v, rows_v, out_v, sem0, sem1):
    sems = (sem0, sem1)
    wid = lax.axis_index("s") * NC + lax.axis_index("c")
    base_p = wid * PW

    pltpu.sync_copy(w_h.at[wid], w_v)
    pltpu.sync_copy(m_h.at[wid], m_v)

    def start_gather(ci, t):
      pltpu.async_copy(table_h.at[idx_v.at[ci]], rows_v.at[t], sems[t])

    def wait_gather(t):
      pltpu.make_async_copy(
          table_h.at[idx_v.at[0]], rows_v.at[t], sems[t]).wait()

    def compute_chunk(ci, t, b):
      for j in range(P_CHUNK):
        wv = w_v[ci, pl.ds(j * M, M)]
        mv = m_v[ci, pl.ds(j * M, M)]
        pv = jnp.abs(wv) * mv
        s = jnp.sum(pv) + jnp.float32(1e-8)
        pvn = pv / s
        acc = [jnp.zeros((L,), jnp.float32) for _ in range(CCH)]
        for m in range(M):
          ws = pvn[m]
          for cc in range(CCH):
            acc[cc] = acc[cc] + ws * rows_v[t, j * M + m, pl.ds(cc * L, L)]
        for cc in range(CCH):
          out_v[t, j, pl.ds(cc * L, L)] = acc[cc]
      pltpu.sync_copy(
          out_v.at[t],
          out_h.at[b].at[pl.ds(base_p + ci * P_CHUNK, P_CHUNK)])

    @pl.loop(0, B)
    def batch(b):
      pltpu.sync_copy(idx_h.at[b].at[wid], idx_v)
      for t in range(NBUF):
        start_gather(t, t)

      @pl.loop(0, NCH // NBUF)
      def body(i):
        ci0 = i * NBUF
        for t in range(NBUF):
          wait_gather(t)
          compute_chunk(ci0 + t, t, b)
          start_gather(jnp.minimum(ci0 + t + NBUF, NCH - 1), t)

      for t in range(NBUF):
        wait_gather(t)

  return k(table, idx_chunks, w_chunks, m_chunks)


def kernel(in_pc_pad, neighbor_id_lstlst, neighbor_mask_lst, p_neighbors):
  B, IN_PN, C = in_pc_pad.shape
  OUT_PN, M = p_neighbors.shape
  assert M == L and C % L == 0

  P_pad = ((OUT_PN + NW * P_CHUNK - 1) // (NW * P_CHUNK)) * (NW * P_CHUNK)
  pad = P_pad - OUT_PN
  pw = P_pad // NW
  nch = pw // P_CHUNK

  ids = neighbor_id_lstlst.astype(jnp.int32)
  ids = jnp.pad(ids, ((0, pad), (0, 0)))
  w_pad = jnp.pad(p_neighbors, ((0, pad), (0, 0)))
  m_pad = jnp.pad(neighbor_mask_lst, ((0, pad), (0, 0)))

  offs = (jnp.arange(B, dtype=jnp.int32) * IN_PN)[:, None, None]
  idx_chunks = (ids[None] + offs).reshape(B, NW, nch, P_CHUNK * M)
  w_chunks = w_pad.reshape(NW, nch, P_CHUNK * M)
  m_chunks = m_pad.reshape(NW, nch, P_CHUNK * M)
  table = in_pc_pad.reshape(B * IN_PN, C)

  out = _pooling_sc(table, idx_chunks, w_chunks, m_chunks,
                    B=B, P_pad=P_pad, C=C, M=M)
  return out[:, :OUT_PN, :]
